# baseline (speedup 1.0000x reference)
import functools

import jax
import jax.numpy as jnp
from jax import lax
from jax.experimental import pallas as pl
from jax.experimental.pallas import tpu as pltpu
from jax.experimental.pallas import tpu_sc as plsc

N_WORKERS = 32
NPW = 320
NPAD = N_WORKERS * NPW
EB = 128
OC = 64
LANES = 16


def _make_agg_kernel(D, E):
    nv = D // LANES
    nw = D // (2 * LANES)
    mesh = plsc.VectorSubcoreMesh(core_axis_name="c", subcore_axis_name="s")

    @functools.partial(
        pl.kernel,
        mesh=mesh,
        out_type=jax.ShapeDtypeStruct((NPAD, D), jnp.float32),
        scratch_types=[
            pltpu.VMEM((NPW + 16,), jnp.int32),
            pltpu.VMEM((2 * EB,), jnp.int32),
            pltpu.VMEM((2 * EB, D // 2), jnp.int32),
            pltpu.VMEM((OC, D), jnp.float32),
            pltpu.SemaphoreType.DMA,
            pltpu.SemaphoreType.DMA,
            pltpu.SemaphoreType.DMA,
            pltpu.SemaphoreType.DMA,
        ],
    )
    def agg(x_hbm, ptr_hbm, idx_hbm, out_hbm, ptr_v, idx_v, rows_v, out_v,
            si0, si1, sr0, sr1):
        wid = lax.axis_index("s") * 2 + lax.axis_index("c")
        base = wid * NPW
        pltpu.sync_copy(ptr_hbm.at[pl.ds(base, NPW + 16)], ptr_v)

        e0 = ptr_v[pl.ds(0, LANES)][0]
        e0a = e0 - jnp.bitwise_and(e0, 7)
        e0a = pl.multiple_of(e0a, 8)

        idx_slc = (idx_v.at[pl.ds(0, EB)], idx_v.at[pl.ds(EB, EB)])
        row_slc = (rows_v.at[pl.ds(0, EB)], rows_v.at[pl.ds(EB, EB)])
        sis = (si0, si1)
        srs = (sr0, sr1)

        def idx_copy(b, par):
            pltpu.async_copy(
                idx_hbm.at[pl.ds(e0a + b * EB, EB)], idx_slc[par], sis[par]
            )

        def idx_wait(b, par):
            pltpu.make_async_copy(
                idx_hbm.at[pl.ds(e0a + b * EB, EB)], idx_slc[par], sis[par]
            ).wait()

        def row_gather(par):
            pltpu.async_copy(x_hbm.at[idx_slc[par]], row_slc[par], srs[par])

        def row_wait(par):
            pltpu.make_async_copy(
                x_hbm.at[idx_slc[par]], row_slc[par], srs[par]
            ).wait()

        idx_copy(0, 0)
        idx_copy(1, 1)
        idx_wait(0, 0)
        row_gather(0)

        zero = jnp.zeros((LANES,), jnp.float32)

        def node_body(n, loaded):
            pv = ptr_v[pl.ds(n, LANES)]
            s = pv[0]
            t = pv[1]
            b_lo = lax.shift_right_logical(s - e0a, 7)
            b_hi = jnp.where(
                t > s, lax.shift_right_logical(t - 1 - e0a, 7) + 1, b_lo
            )

            @pl.loop(b_lo, b_hi, init_carry=(loaded, (zero,) * nv))
            def batch_loop(b, carry):
                loaded, acc = carry
                par = jnp.bitwise_and(b, 1)

                @pl.when(b != loaded)
                def _():
                    @pl.when(par == 0)
                    def _():
                        row_wait(0)
                        idx_wait(1, 1)
                        idx_copy(b + 2, 0)
                        row_gather(1)

                    @pl.when(par == 1)
                    def _():
                        row_wait(1)
                        idx_wait(0, 0)
                        idx_copy(b + 2, 1)
                        row_gather(0)

                bs = e0a + b * EB
                el = jnp.maximum(s, bs)
                eh = jnp.minimum(t, bs + EB)
                off = par * EB - bs

                mask_hi = jnp.full((LANES,), -65536, jnp.int32)
                sh16 = jnp.full((LANES,), 16, jnp.int32)

                @pl.loop(el, eh, init_carry=acc)
                def edge_loop(e, acc):
                    pos = e + off
                    nacc = []
                    for j in range(nw):
                        v = rows_v[pos, pl.ds(j * LANES, LANES)]
                        even = lax.bitcast_convert_type(
                            lax.shift_left(v, sh16), jnp.float32
                        )
                        odd = lax.bitcast_convert_type(
                            jnp.bitwise_and(v, mask_hi), jnp.float32
                        )
                        nacc.append(acc[2 * j] + even)
                        nacc.append(acc[2 * j + 1] + odd)
                    return tuple(nacc)

                return (b, edge_loop)

            loaded, acc = batch_loop
            cnt = jnp.broadcast_to(
                jnp.maximum(t - s, 1), (LANES,)
            ).astype(jnp.float32)
            scale = jnp.ones((LANES,), jnp.float32) / cnt
            slot = jnp.bitwise_and(n, OC - 1)
            for k in range(nv):
                out_v[slot, pl.ds(k * LANES, LANES)] = acc[k] * scale

            @pl.when(slot == OC - 1)
            def _():
                dst = pl.multiple_of(base + n - (OC - 1), OC)
                pltpu.sync_copy(out_v, out_hbm.at[pl.ds(dst, OC)])

            return loaded

        loaded = lax.fori_loop(0, NPW, node_body, jnp.int32(-1))

        lpar = jnp.bitwise_and(loaded, 1)

        @pl.when(lpar == 0)
        def _():
            idx_wait(loaded + 2, 0)
            row_wait(1)

        @pl.when(lpar == 1)
        def _():
            idx_wait(loaded + 2, 1)
            row_wait(0)

    return agg


def _mm_kernel(agg_ref, x_ref, wl_ref, wr_ref, b_ref, o_ref):
    o_ref[...] = (
        jnp.dot(agg_ref[...], wl_ref[...], preferred_element_type=jnp.float32)
        + jnp.dot(x_ref[...], wr_ref[...], preferred_element_type=jnp.float32)
        + b_ref[...]
    )


def kernel(x, ptr, idx, num_node, W_l, b_l, W_r):
    N, D = x.shape
    H = W_l.shape[1]
    E = idx.shape[0]

    ptr = ptr.astype(jnp.int32)
    idx = idx.astype(jnp.int32)
    ptr_pad = jnp.concatenate(
        [ptr, jnp.full((NPAD + 16 - (N + 1),), ptr[-1], jnp.int32)]
    )
    idx_pad = jnp.concatenate([idx, jnp.zeros((4 * EB + 8,), jnp.int32)])

    x_pk = lax.bitcast_convert_type(
        x.astype(jnp.bfloat16).reshape(N, D // 2, 2), jnp.int32
    )

    agg = _make_agg_kernel(D, E)(x_pk, ptr_pad, idx_pad)

    qq = jnp.arange(D // LANES)[:, None]
    ll = jnp.arange(LANES)[None, :]
    perm = (32 * (qq // 2) + 2 * ll + (qq % 2)).reshape(-1)
    W_l = W_l[perm, :]

    x_pad = jnp.pad(x, ((0, NPAD - N), (0, 0)))
    b2 = b_l.reshape(1, H)

    BN = 1024
    out = pl.pallas_call(
        _mm_kernel,
        grid=(NPAD // BN,),
        in_specs=[
            pl.BlockSpec((BN, D), lambda i: (i, 0)),
            pl.BlockSpec((BN, D), lambda i: (i, 0)),
            pl.BlockSpec((D, H), lambda i: (0, 0)),
            pl.BlockSpec((D, H), lambda i: (0, 0)),
            pl.BlockSpec((1, H), lambda i: (0, 0)),
        ],
        out_specs=pl.BlockSpec((BN, H), lambda i: (i, 0)),
        out_shape=jax.ShapeDtypeStruct((NPAD, H), jnp.float32),
    )(agg, x_pad, W_l, W_r, b2)

    return out[:N]


# speedup vs baseline: 1.6745x; 1.6745x over previous
import functools

import jax
import jax.numpy as jnp
from jax import lax
from jax.experimental import pallas as pl
from jax.experimental.pallas import tpu as pltpu
from jax.experimental.pallas import tpu_sc as plsc

N_WORKERS = 32
NPW = 320
NPAD = N_WORKERS * NPW
EB = 128
OC = 64
LANES = 16


def _make_pack_kernel(N, D):
    CH = 80
    nch_full = NPW // CH
    rows_last = N - NPW * (N_WORKERS - 1)
    nch_last = (rows_last + CH - 1) // CH
    assert rows_last % CH == 0 and N % CH == 0
    mesh = plsc.VectorSubcoreMesh(core_axis_name="c", subcore_axis_name="s")

    @functools.partial(
        pl.kernel,
        mesh=mesh,
        out_type=jax.ShapeDtypeStruct((N, D // 2), jnp.int32),
        scratch_types=[
            pltpu.VMEM((CH, D), jnp.float32),
            pltpu.VMEM((CH, D // 2), jnp.int32),
        ],
    )
    def pack(x_hbm, out_hbm, buf_in, buf_out):
        wid = lax.axis_index("s") * 2 + lax.axis_index("c")
        base = wid * NPW
        nch = jnp.where(wid < N_WORKERS - 1, nch_full, nch_last)

        half = jnp.full((LANES,), 0x8000, jnp.int32)
        mask_hi = jnp.full((LANES,), -65536, jnp.int32)
        sh16 = jnp.full((LANES,), 16, jnp.int32)

        @pl.loop(0, nch)
        def chunk_loop(c):
            src = pl.multiple_of(base + c * CH, 8)
            pltpu.sync_copy(x_hbm.at[pl.ds(src, CH)], buf_in)

            @pl.loop(0, CH)
            def row_loop(r):
                for m in range(D // (2 * LANES)):
                    lo = buf_in[r, pl.ds(2 * m * LANES, LANES)]
                    hi = buf_in[r, pl.ds((2 * m + 1) * LANES, LANES)]
                    li = lax.bitcast_convert_type(lo, jnp.int32) + half
                    hi_i = lax.bitcast_convert_type(hi, jnp.int32) + half
                    pk = jnp.bitwise_or(
                        lax.shift_right_logical(li, sh16),
                        jnp.bitwise_and(hi_i, mask_hi),
                    )
                    buf_out[r, pl.ds(m * LANES, LANES)] = pk
            pltpu.sync_copy(buf_out, out_hbm.at[pl.ds(src, CH)])

    return pack


def _make_agg_kernel(D, E):
    nv = D // LANES
    nw = D // (2 * LANES)
    mesh = plsc.VectorSubcoreMesh(core_axis_name="c", subcore_axis_name="s")

    @functools.partial(
        pl.kernel,
        mesh=mesh,
        out_type=jax.ShapeDtypeStruct((NPAD, D), jnp.float32),
        scratch_types=[
            pltpu.VMEM((NPW + 16,), jnp.int32),
            pltpu.VMEM((2 * EB,), jnp.int32),
            pltpu.VMEM((2 * EB, D // 2), jnp.int32),
            pltpu.VMEM((OC, D), jnp.float32),
            pltpu.SemaphoreType.DMA,
            pltpu.SemaphoreType.DMA,
            pltpu.SemaphoreType.DMA,
            pltpu.SemaphoreType.DMA,
        ],
    )
    def agg(x_hbm, ptr_hbm, idx_hbm, out_hbm, ptr_v, idx_v, rows_v, out_v,
            si0, si1, sr0, sr1):
        wid = lax.axis_index("s") * 2 + lax.axis_index("c")
        base = wid * NPW
        pltpu.sync_copy(ptr_hbm.at[pl.ds(base, NPW + 16)], ptr_v)

        e0 = ptr_v[pl.ds(0, LANES)][0]
        e0a = e0 - jnp.bitwise_and(e0, 7)
        e0a = pl.multiple_of(e0a, 8)

        idx_slc = (idx_v.at[pl.ds(0, EB)], idx_v.at[pl.ds(EB, EB)])
        row_slc = (rows_v.at[pl.ds(0, EB)], rows_v.at[pl.ds(EB, EB)])
        sis = (si0, si1)
        srs = (sr0, sr1)

        def idx_copy(b, par):
            pltpu.async_copy(
                idx_hbm.at[pl.ds(e0a + b * EB, EB)], idx_slc[par], sis[par]
            )

        def idx_wait(b, par):
            pltpu.make_async_copy(
                idx_hbm.at[pl.ds(e0a + b * EB, EB)], idx_slc[par], sis[par]
            ).wait()

        def row_gather(par):
            pltpu.async_copy(x_hbm.at[idx_slc[par]], row_slc[par], srs[par])

        def row_wait(par):
            pltpu.make_async_copy(
                x_hbm.at[idx_slc[par]], row_slc[par], srs[par]
            ).wait()

        idx_copy(0, 0)
        idx_copy(1, 1)
        idx_wait(0, 0)
        row_gather(0)

        zero = jnp.zeros((LANES,), jnp.float32)

        def node_body(n, loaded):
            pv = ptr_v[pl.ds(n, LANES)]
            s = pv[0]
            t = pv[1]
            b_lo = lax.shift_right_logical(s - e0a, 7)
            b_hi = jnp.where(
                t > s, lax.shift_right_logical(t - 1 - e0a, 7) + 1, b_lo
            )

            @pl.loop(b_lo, b_hi, init_carry=(loaded, (zero,) * nv))
            def batch_loop(b, carry):
                loaded, acc = carry
                par = jnp.bitwise_and(b, 1)

                @pl.when(b != loaded)
                def _():
                    @pl.when(par == 0)
                    def _():
                        row_wait(0)
                        idx_wait(1, 1)
                        idx_copy(b + 2, 0)
                        row_gather(1)

                    @pl.when(par == 1)
                    def _():
                        row_wait(1)
                        idx_wait(0, 0)
                        idx_copy(b + 2, 1)
                        row_gather(0)

                bs = e0a + b * EB
                el = jnp.maximum(s, bs)
                eh = jnp.minimum(t, bs + EB)
                off = par * EB - bs

                mask_hi = jnp.full((LANES,), -65536, jnp.int32)
                sh16 = jnp.full((LANES,), 16, jnp.int32)

                @pl.loop(el, eh, init_carry=acc)
                def edge_loop(e, acc):
                    pos = e + off
                    nacc = []
                    for j in range(nw):
                        v = rows_v[pos, pl.ds(j * LANES, LANES)]
                        even = lax.bitcast_convert_type(
                            lax.shift_left(v, sh16), jnp.float32
                        )
                        odd = lax.bitcast_convert_type(
                            jnp.bitwise_and(v, mask_hi), jnp.float32
                        )
                        nacc.append(acc[2 * j] + even)
                        nacc.append(acc[2 * j + 1] + odd)
                    return tuple(nacc)

                return (b, edge_loop)

            loaded, acc = batch_loop
            cnt = jnp.broadcast_to(
                jnp.maximum(t - s, 1), (LANES,)
            ).astype(jnp.float32)
            scale = jnp.ones((LANES,), jnp.float32) / cnt
            slot = jnp.bitwise_and(n, OC - 1)
            for k in range(nv):
                out_v[slot, pl.ds(k * LANES, LANES)] = acc[k] * scale

            @pl.when(slot == OC - 1)
            def _():
                dst = pl.multiple_of(base + n - (OC - 1), OC)
                pltpu.sync_copy(out_v, out_hbm.at[pl.ds(dst, OC)])

            return loaded

        loaded = lax.fori_loop(0, NPW, node_body, jnp.int32(-1))

        lpar = jnp.bitwise_and(loaded, 1)

        @pl.when(lpar == 0)
        def _():
            idx_wait(loaded + 2, 0)
            row_wait(1)

        @pl.when(lpar == 1)
        def _():
            idx_wait(loaded + 2, 1)
            row_wait(0)

    return agg


def _mm_kernel(agg_ref, x_ref, wl_ref, wr_ref, b_ref, o_ref):
    o_ref[...] = (
        jnp.dot(agg_ref[...], wl_ref[...], preferred_element_type=jnp.float32)
        + jnp.dot(x_ref[...], wr_ref[...], preferred_element_type=jnp.float32)
        + b_ref[...]
    )


def kernel(x, ptr, idx, num_node, W_l, b_l, W_r):
    N, D = x.shape
    H = W_l.shape[1]
    E = idx.shape[0]

    ptr = ptr.astype(jnp.int32)
    idx = idx.astype(jnp.int32)
    ptr_pad = jnp.concatenate(
        [ptr, jnp.full((NPAD + 16 - (N + 1),), ptr[-1], jnp.int32)]
    )
    idx_pad = jnp.concatenate([idx, jnp.zeros((4 * EB + 8,), jnp.int32)])

    x_pk = _make_pack_kernel(N, D)(x)

    agg = _make_agg_kernel(D, E)(x_pk, ptr_pad, idx_pad)

    b2 = b_l.reshape(1, H)

    BN = 1000
    out = pl.pallas_call(
        _mm_kernel,
        grid=(N // BN,),
        in_specs=[
            pl.BlockSpec((BN, D), lambda i: (i, 0)),
            pl.BlockSpec((BN, D), lambda i: (i, 0)),
            pl.BlockSpec((D, H), lambda i: (0, 0)),
            pl.BlockSpec((D, H), lambda i: (0, 0)),
            pl.BlockSpec((1, H), lambda i: (0, 0)),
        ],
        out_specs=pl.BlockSpec((BN, H), lambda i: (i, 0)),
        out_shape=jax.ShapeDtypeStruct((N, H), jnp.float32),
    )(agg, x, W_l, W_r, b2)

    return out


# speedup vs baseline: 1.6807x; 1.0037x over previous
import functools

import jax
import jax.numpy as jnp
from jax import lax
from jax.experimental import pallas as pl
from jax.experimental.pallas import tpu as pltpu
from jax.experimental.pallas import tpu_sc as plsc

N_WORKERS = 32
NPW = 320
NPAD = N_WORKERS * NPW
EB = 128
OC = 64
LANES = 16


def _make_pack_kernel(N, D):
    CH = 80
    nch_full = NPW // CH
    rows_last = N - NPW * (N_WORKERS - 1)
    nch_last = (rows_last + CH - 1) // CH
    assert rows_last % CH == 0 and N % CH == 0
    mesh = plsc.VectorSubcoreMesh(core_axis_name="c", subcore_axis_name="s")

    @functools.partial(
        pl.kernel,
        mesh=mesh,
        out_type=jax.ShapeDtypeStruct((N, D // 2), jnp.int32),
        scratch_types=[
            pltpu.VMEM((CH, D), jnp.float32),
            pltpu.VMEM((CH, D // 2), jnp.int32),
        ],
    )
    def pack(x_hbm, out_hbm, buf_in, buf_out):
        wid = lax.axis_index("s") * 2 + lax.axis_index("c")
        base = wid * NPW
        nch = jnp.where(wid < N_WORKERS - 1, nch_full, nch_last)

        half = jnp.full((LANES,), 0x8000, jnp.int32)
        mask_hi = jnp.full((LANES,), -65536, jnp.int32)
        sh16 = jnp.full((LANES,), 16, jnp.int32)

        @pl.loop(0, nch)
        def chunk_loop(c):
            src = pl.multiple_of(base + c * CH, 8)
            pltpu.sync_copy(x_hbm.at[pl.ds(src, CH)], buf_in)

            @pl.loop(0, CH)
            def row_loop(r):
                for m in range(D // (2 * LANES)):
                    lo = buf_in[r, pl.ds(2 * m * LANES, LANES)]
                    hi = buf_in[r, pl.ds((2 * m + 1) * LANES, LANES)]
                    li = lax.bitcast_convert_type(lo, jnp.int32) + half
                    hi_i = lax.bitcast_convert_type(hi, jnp.int32) + half
                    pk = jnp.bitwise_or(
                        lax.shift_right_logical(li, sh16),
                        jnp.bitwise_and(hi_i, mask_hi),
                    )
                    buf_out[r, pl.ds(m * LANES, LANES)] = pk
            pltpu.sync_copy(buf_out, out_hbm.at[pl.ds(src, CH)])

    return pack


def _make_agg_kernel(D, E):
    nv = D // LANES
    nw = D // (2 * LANES)
    mesh = plsc.VectorSubcoreMesh(core_axis_name="c", subcore_axis_name="s")

    @functools.partial(
        pl.kernel,
        mesh=mesh,
        out_type=jax.ShapeDtypeStruct((NPAD, D), jnp.float32),
        scratch_types=[
            pltpu.VMEM((NPW + 16,), jnp.int32),
            pltpu.VMEM((2 * EB,), jnp.int32),
            pltpu.VMEM((2 * EB, D // 2), jnp.int32),
            pltpu.VMEM((OC, D), jnp.float32),
            pltpu.SemaphoreType.DMA,
            pltpu.SemaphoreType.DMA,
            pltpu.SemaphoreType.DMA,
            pltpu.SemaphoreType.DMA,
        ],
    )
    def agg(x_hbm, ptr_hbm, idx_hbm, out_hbm, ptr_v, idx_v, rows_v, out_v,
            si0, si1, sr0, sr1):
        wid = lax.axis_index("s") * 2 + lax.axis_index("c")
        base = wid * NPW
        pltpu.sync_copy(ptr_hbm.at[pl.ds(base, NPW + 16)], ptr_v)

        e0 = ptr_v[pl.ds(0, LANES)][0]
        e0a = e0 - jnp.bitwise_and(e0, 7)
        e0a = pl.multiple_of(e0a, 8)

        idx_slc = (idx_v.at[pl.ds(0, EB)], idx_v.at[pl.ds(EB, EB)])
        row_slc = (rows_v.at[pl.ds(0, EB)], rows_v.at[pl.ds(EB, EB)])
        sis = (si0, si1)
        srs = (sr0, sr1)

        def idx_copy(b, par):
            pltpu.async_copy(
                idx_hbm.at[pl.ds(e0a + b * EB, EB)], idx_slc[par], sis[par]
            )

        def idx_wait(b, par):
            pltpu.make_async_copy(
                idx_hbm.at[pl.ds(e0a + b * EB, EB)], idx_slc[par], sis[par]
            ).wait()

        def row_gather(par):
            pltpu.async_copy(x_hbm.at[idx_slc[par]], row_slc[par], srs[par])

        def row_wait(par):
            pltpu.make_async_copy(
                x_hbm.at[idx_slc[par]], row_slc[par], srs[par]
            ).wait()

        idx_copy(0, 0)
        idx_copy(1, 1)
        idx_wait(0, 0)
        row_gather(0)

        zero = jnp.zeros((LANES,), jnp.float32)

        def node_body(n, loaded):
            pv = ptr_v[pl.ds(n, LANES)]
            s = pv[0]
            t = pv[1]
            b_lo = lax.shift_right_logical(s - e0a, 7)
            b_hi = jnp.where(
                t > s, lax.shift_right_logical(t - 1 - e0a, 7) + 1, b_lo
            )

            @pl.loop(b_lo, b_hi, init_carry=(loaded, (zero,) * nv))
            def batch_loop(b, carry):
                loaded, acc = carry
                par = jnp.bitwise_and(b, 1)

                @pl.when(b != loaded)
                def _():
                    @pl.when(par == 0)
                    def _():
                        row_wait(0)
                        idx_wait(1, 1)
                        idx_copy(b + 2, 0)
                        row_gather(1)

                    @pl.when(par == 1)
                    def _():
                        row_wait(1)
                        idx_wait(0, 0)
                        idx_copy(b + 2, 1)
                        row_gather(0)

                bs = e0a + b * EB
                el = jnp.maximum(s, bs)
                eh = jnp.minimum(t, bs + EB)
                off = par * EB - bs

                mask_hi = jnp.full((LANES,), -65536, jnp.int32)
                sh16 = jnp.full((LANES,), 16, jnp.int32)

                @pl.loop(el, eh, init_carry=acc)
                def edge_loop(e, acc):
                    pos = e + off
                    nacc = []
                    for j in range(nw):
                        v = rows_v[pos, pl.ds(j * LANES, LANES)]
                        even = lax.bitcast_convert_type(
                            lax.shift_left(v, sh16), jnp.float32
                        )
                        odd = lax.bitcast_convert_type(
                            jnp.bitwise_and(v, mask_hi), jnp.float32
                        )
                        nacc.append(acc[2 * j] + even)
                        nacc.append(acc[2 * j + 1] + odd)
                    return tuple(nacc)

                return (b, edge_loop)

            loaded, acc = batch_loop
            cnt = jnp.broadcast_to(
                jnp.maximum(t - s, 1), (LANES,)
            ).astype(jnp.float32)
            scale = jnp.ones((LANES,), jnp.float32) / cnt
            slot = jnp.bitwise_and(n, OC - 1)
            for k in range(nv):
                out_v[slot, pl.ds(k * LANES, LANES)] = acc[k] * scale

            @pl.when(slot == OC - 1)
            def _():
                dst = pl.multiple_of(base + n - (OC - 1), OC)
                pltpu.sync_copy(out_v, out_hbm.at[pl.ds(dst, OC)])

            return loaded

        loaded = lax.fori_loop(0, NPW, node_body, jnp.int32(-1))

        lpar = jnp.bitwise_and(loaded, 1)

        @pl.when(lpar == 0)
        def _():
            idx_wait(loaded + 2, 0)
            row_wait(1)

        @pl.when(lpar == 1)
        def _():
            idx_wait(loaded + 2, 1)
            row_wait(0)

    return agg


def _mm_kernel(agg_ref, x_ref, wl_ref, wr_ref, b_ref, o_ref):
    a = agg_ref[...].astype(jnp.bfloat16)
    xb = x_ref[...].astype(jnp.bfloat16)
    o_ref[...] = (
        jnp.dot(a, wl_ref[...], preferred_element_type=jnp.float32)
        + jnp.dot(xb, wr_ref[...], preferred_element_type=jnp.float32)
        + b_ref[...]
    )


def kernel(x, ptr, idx, num_node, W_l, b_l, W_r):
    N, D = x.shape
    H = W_l.shape[1]
    E = idx.shape[0]

    ptr = ptr.astype(jnp.int32)
    idx = idx.astype(jnp.int32)
    ptr_pad = jnp.concatenate(
        [ptr, jnp.full((NPAD + 16 - (N + 1),), ptr[-1], jnp.int32)]
    )
    idx_pad = jnp.concatenate([idx, jnp.zeros((4 * EB + 8,), jnp.int32)])

    x_pk = _make_pack_kernel(N, D)(x)

    agg = _make_agg_kernel(D, E)(x_pk, ptr_pad, idx_pad)

    b2 = b_l.reshape(1, H)

    BN = 1000
    out = pl.pallas_call(
        _mm_kernel,
        grid=(N // BN,),
        in_specs=[
            pl.BlockSpec((BN, D), lambda i: (i, 0)),
            pl.BlockSpec((BN, D), lambda i: (i, 0)),
            pl.BlockSpec((D, H), lambda i: (0, 0)),
            pl.BlockSpec((D, H), lambda i: (0, 0)),
            pl.BlockSpec((1, H), lambda i: (0, 0)),
        ],
        out_specs=pl.BlockSpec((BN, H), lambda i: (i, 0)),
        out_shape=jax.ShapeDtypeStruct((N, H), jnp.float32),
    )(agg, x, W_l.astype(jnp.bfloat16), W_r.astype(jnp.bfloat16), b2)

    return out
